# Initial kernel scaffold; baseline (speedup 1.0000x reference)
import jax
import jax.numpy as jnp
from jax.experimental import pallas as pl


def kernel(x, edge_index, edge_attr, W1, root1, b1, W2, root2, b2, W3, root3, b3, fc_w, fc_b):
    raise NotImplementedError("write your pallas kernel here")


# speedup vs baseline: 5.0892x; 5.0892x over previous
import functools

import jax
import jax.numpy as jnp
from jax import lax
from jax.experimental import pallas as pl
from jax.experimental.pallas import tpu as pltpu
from jax.experimental.pallas import tpu_sc as plsc

N = 10000
E = 320000
D = 128
R = 16
SEG = N * R

NC = 2
NS = 16
NW = NC * NS
EPW = E // NW
CH = 80
NCH = EPW // CH
SPW = SEG // NS
ZR = 200
ZSPAN = N // ZR
WB = 1000

_MESH = plsc.VectorSubcoreMesh(core_axis_name="c", subcore_axis_name="s")


def _z16():
    return jnp.zeros((16,), jnp.float32)


def _o16():
    return jnp.ones((16,), jnp.float32)


def _wid(c, s):
    return s * NC + c


def _cnt_body(comb_hbm, out_hbm, idx_v, ones_v, zero_v, cnt_sh):
    c = lax.axis_index("c")
    s = lax.axis_index("s")
    w = _wid(c, s)

    def _z(i, _):
        zero_v[pl.ds(i * 16, 16)] = _z16()
        return 0

    lax.fori_loop(0, SPW // 16, _z, 0)
    for k in range(CH // 16):
        ones_v[pl.ds(k * 16, 16)] = _o16()
    pltpu.sync_copy(zero_v, cnt_sh.at[pl.ds(s * SPW, SPW)])
    plsc.subcore_barrier()

    def _step(i, _):
        base = w * EPW + i * CH
        pltpu.sync_copy(comb_hbm.at[pl.ds(base, CH)], idx_v)
        pltpu.sync_copy(ones_v, cnt_sh.at[idx_v], add=True)
        return 0

    lax.fori_loop(0, NCH, _step, 0)
    plsc.subcore_barrier()
    pltpu.sync_copy(cnt_sh.at[pl.ds(s * SPW, SPW)], zero_v)
    pltpu.sync_copy(zero_v, out_hbm.at[pl.ds(c * SEG + s * SPW, SPW)])


_cnt_call = functools.partial(
    pl.kernel,
    out_type=jax.ShapeDtypeStruct((NC * SEG,), jnp.float32),
    mesh=_MESH,
    scratch_types=[
        pltpu.VMEM((CH,), jnp.int32),
        pltpu.VMEM((CH,), jnp.float32),
        pltpu.VMEM((SPW,), jnp.float32),
        pltpu.VMEM_SHARED((SEG,), jnp.float32),
    ],
)(_cnt_body)


def _norm_body(cnt_hbm, comb_hbm, w_hbm, idx_v, c_v, w_v, sem):
    c = lax.axis_index("c")
    s = lax.axis_index("s")
    w = _wid(c, s)

    def _step(i, _):
        base = w * EPW + i * CH
        pltpu.sync_copy(comb_hbm.at[pl.ds(base, CH)], idx_v)
        pltpu.async_copy(cnt_hbm.at[idx_v], c_v, sem).wait()
        for k in range(CH // 16):
            cv = c_v[pl.ds(k * 16, 16)]
            w_v[pl.ds(k * 16, 16)] = 1.0 / jnp.maximum(cv, 1.0)
        pltpu.sync_copy(w_v, w_hbm.at[pl.ds(base, CH)])
        return 0

    lax.fori_loop(0, NCH, _step, 0)


_norm_call = functools.partial(
    pl.kernel,
    out_type=jax.ShapeDtypeStruct((E,), jnp.float32),
    mesh=_MESH,
    scratch_types=[
        pltpu.VMEM((CH,), jnp.int32),
        pltpu.VMEM((CH,), jnp.float32),
        pltpu.VMEM((CH,), jnp.float32),
        pltpu.SemaphoreType.DMA,
    ],
)(_norm_body)


def _agg_body(allx_hbm, eidx_hbm, dst_hbm, w_hbm, out_hbm,
              ei_v, dst_v, w_v, rows_v, zrow_v, agg_sh, sem):
    c = lax.axis_index("c")
    s = lax.axis_index("s")
    w = _wid(c, s)

    def _zr(i, _):
        for j in range(D // 16):
            zrow_v[i, pl.ds(j * 16, 16)] = _z16()
        return 0

    lax.fori_loop(0, ZR, _zr, 0)
    for k in range(ZSPAN // NS + 1):
        r = s + NS * k

        @pl.when(r < ZSPAN)
        def _():
            pltpu.sync_copy(zrow_v, agg_sh.at[pl.ds(r * ZR, ZR)])

    plsc.subcore_barrier()

    def _step(i, _):
        base = w * EPW + i * CH
        pltpu.sync_copy(eidx_hbm.at[pl.ds(base, CH)], ei_v)
        pltpu.sync_copy(dst_hbm.at[pl.ds(base, CH)], dst_v)
        pltpu.sync_copy(w_hbm.at[pl.ds(base, CH)], w_v)
        pltpu.async_copy(allx_hbm.at[ei_v], rows_v, sem).wait()

        def _scale(g, _):
            wv = w_v[pl.ds(g * 16, 16)]
            row0 = g * 16
            for b in range(16):
                wb = wv[b]
                for j in range(D // 16):
                    rows_v[row0 + b, pl.ds(j * 16, 16)] = (
                        rows_v[row0 + b, pl.ds(j * 16, 16)] * wb)
            return 0

        lax.fori_loop(0, CH // 16, _scale, 0)
        pltpu.sync_copy(rows_v, agg_sh.at[dst_v], add=True)
        return 0

    lax.fori_loop(0, NCH, _step, 0)
    plsc.subcore_barrier()

    @pl.when(s < N // WB)
    def _():
        for k in range(WB // ZR):
            row0 = s * WB + k * ZR
            pltpu.sync_copy(agg_sh.at[pl.ds(row0, ZR)], zrow_v)
            pltpu.sync_copy(zrow_v, out_hbm.at[c, pl.ds(row0, ZR)])


_agg_call = functools.partial(
    pl.kernel,
    out_type=jax.ShapeDtypeStruct((NC, N, D), jnp.float32),
    mesh=_MESH,
    scratch_types=[
        pltpu.VMEM((CH,), jnp.int32),
        pltpu.VMEM((CH,), jnp.int32),
        pltpu.VMEM((CH,), jnp.float32),
        pltpu.VMEM((CH, D), jnp.float32),
        pltpu.VMEM((ZR, D), jnp.float32),
        pltpu.VMEM_SHARED((N, D), jnp.float32),
        pltpu.SemaphoreType.DMA,
    ],
)(_agg_body)


BLK = 1000
NBLK = N // BLK


def _first_body(x_ref, w_ref, ax_ref):
    xb = x_ref[...]
    for r in range(R):
        ax_ref[r] = jnp.dot(xb, w_ref[r], preferred_element_type=jnp.float32)


def _tc_first(x, W):
    out = pl.pallas_call(
        _first_body,
        grid=(NBLK,),
        in_specs=[
            pl.BlockSpec((BLK, D), lambda i: (i, 0)),
            pl.BlockSpec((R, D, D), lambda i: (0, 0, 0)),
        ],
        out_specs=pl.BlockSpec((R, BLK, D), lambda i: (0, i, 0)),
        out_shape=jax.ShapeDtypeStruct((R, N, D), jnp.float32),
    )(x, W)
    return out.reshape(SEG, D)


def _mid_body(p_ref, xp_ref, root_ref, b_ref, w_ref, h_ref, ax_ref):
    agg = p_ref[0] + p_ref[1]
    h = jnp.maximum(
        agg + jnp.dot(xp_ref[...], root_ref[...],
                      preferred_element_type=jnp.float32) + b_ref[...], 0.0)
    h_ref[...] = h
    for r in range(R):
        ax_ref[r] = jnp.dot(h, w_ref[r], preferred_element_type=jnp.float32)


def _tc_mid(parts, xp, root, b, Wn):
    h, ax = pl.pallas_call(
        _mid_body,
        grid=(NBLK,),
        in_specs=[
            pl.BlockSpec((NC, BLK, D), lambda i: (0, i, 0)),
            pl.BlockSpec((BLK, D), lambda i: (i, 0)),
            pl.BlockSpec((D, D), lambda i: (0, 0)),
            pl.BlockSpec((1, D), lambda i: (0, 0)),
            pl.BlockSpec((R, D, D), lambda i: (0, 0, 0)),
        ],
        out_specs=[
            pl.BlockSpec((BLK, D), lambda i: (i, 0)),
            pl.BlockSpec((R, BLK, D), lambda i: (0, i, 0)),
        ],
        out_shape=[
            jax.ShapeDtypeStruct((N, D), jnp.float32),
            jax.ShapeDtypeStruct((R, N, D), jnp.float32),
        ],
    )(parts, xp, root, b.reshape(1, D), Wn)
    return h, ax.reshape(SEG, D)


def _last_body(p_ref, xp_ref, root_ref, b_ref, fw_ref, o_ref):
    agg = p_ref[0] + p_ref[1]
    h = jnp.maximum(
        agg + jnp.dot(xp_ref[...], root_ref[...],
                      preferred_element_type=jnp.float32) + b_ref[...], 0.0)
    o_ref[...] = jnp.dot(h, fw_ref[...], preferred_element_type=jnp.float32)


def _tc_last(parts, xp, root, b, fw_pad):
    return pl.pallas_call(
        _last_body,
        grid=(NBLK,),
        in_specs=[
            pl.BlockSpec((NC, BLK, D), lambda i: (0, i, 0)),
            pl.BlockSpec((BLK, D), lambda i: (i, 0)),
            pl.BlockSpec((D, D), lambda i: (0, 0)),
            pl.BlockSpec((1, D), lambda i: (0, 0)),
            pl.BlockSpec((D, D), lambda i: (0, 0)),
        ],
        out_specs=pl.BlockSpec((BLK, D), lambda i: (i, 0)),
        out_shape=jax.ShapeDtypeStruct((N, D), jnp.float32),
    )(parts, xp, root, b.reshape(1, D), fw_pad)


def kernel(x, edge_index, edge_attr, W1, root1, b1, W2, root2, b2,
           W3, root3, b3, fc_w, fc_b):
    src = edge_index[0].astype(jnp.int32)
    dst = edge_index[1].astype(jnp.int32)
    et = edge_attr.astype(jnp.int32)
    comb = dst * R + et
    eidx = et * N + src

    cnt = _cnt_call(comb).reshape(NC, SEG).sum(axis=0)
    w = _norm_call(cnt, comb)

    ax = _tc_first(x, W1)
    parts = _agg_call(ax, eidx, dst, w)
    h1, ax = _tc_mid(parts, x, root1, b1, W2)
    parts = _agg_call(ax, eidx, dst, w)
    h2, ax = _tc_mid(parts, h1, root2, b2, W3)
    parts = _agg_call(ax, eidx, dst, w)
    fw_pad = jnp.pad(fc_w, ((0, 0), (0, D - fc_w.shape[1])))
    out = _tc_last(parts, h2, root3, b3, fw_pad)
    return out[:, :1] + fc_b


# speedup vs baseline: 8.7942x; 1.7280x over previous
import functools

import jax
import jax.numpy as jnp
from jax import lax
from jax.experimental import pallas as pl
from jax.experimental.pallas import tpu as pltpu
from jax.experimental.pallas import tpu_sc as plsc

N = 10000
E = 320000
D = 128
R = 16
SEG = N * R

NC = 2
NS = 16
NW = NC * NS
EPW = E // NW
CH = 80
NCH = EPW // CH
NPAIR = NCH // 2
SPW = SEG // NS
ZB = 2000
ZR = 80
NZB = N // ZR

_MESH = plsc.VectorSubcoreMesh(core_axis_name="c", subcore_axis_name="s")


def _z16():
    return jnp.zeros((16,), jnp.float32)


def _o16():
    return jnp.ones((16,), jnp.float32)


def _wid(c, s):
    return s * NC + c


def _cnt_body(comb_hbm, out_hbm, idx_v, ones_v, zb_v, cnt_sh):
    c = lax.axis_index("c")
    s = lax.axis_index("s")
    w = _wid(c, s)

    def _z(i, _):
        zb_v[pl.ds(i * 16, 16)] = _z16()
        return 0

    lax.fori_loop(0, ZB // 16, _z, 0)
    for k in range(CH // 16):
        ones_v[pl.ds(k * 16, 16)] = _o16()
    for k in range(SPW // ZB):
        pltpu.sync_copy(zb_v, cnt_sh.at[pl.ds(s * SPW + k * ZB, ZB)])
    plsc.subcore_barrier()

    def _step(i, _):
        pltpu.sync_copy(comb_hbm.at[pl.ds(w * EPW + i * CH, CH)], idx_v)
        pltpu.sync_copy(ones_v, cnt_sh.at[idx_v], add=True)
        return 0

    lax.fori_loop(0, NCH, _step, 0)
    plsc.subcore_barrier()
    for k in range(SPW // ZB):
        pltpu.sync_copy(cnt_sh.at[pl.ds(s * SPW + k * ZB, ZB)], zb_v)
        pltpu.sync_copy(zb_v, out_hbm.at[pl.ds(c * SEG + s * SPW + k * ZB,
                                               ZB)])


_cnt_call = functools.partial(
    pl.kernel,
    out_type=jax.ShapeDtypeStruct((NC * SEG,), jnp.float32),
    mesh=_MESH,
    scratch_types=[
        pltpu.VMEM((CH,), jnp.int32),
        pltpu.VMEM((CH,), jnp.float32),
        pltpu.VMEM((ZB,), jnp.float32),
        pltpu.VMEM_SHARED((SEG,), jnp.float32),
    ],
)(_cnt_body)


def _norm_body(cnt_hbm, comb_hbm, w_hbm, idx_v, c_v, wb_v, sem):
    c = lax.axis_index("c")
    s = lax.axis_index("s")
    w = _wid(c, s)

    def _step(i, _):
        base = w * EPW + i * CH
        pltpu.sync_copy(comb_hbm.at[pl.ds(base, CH)], idx_v)
        pltpu.async_copy(cnt_hbm.at[idx_v], c_v, sem).wait()
        for k in range(CH // 16):
            cv = c_v[pl.ds(k * 16, 16)]
            wb_v[pl.ds(k * 16, 16)] = 1.0 / jnp.maximum(cv, 1.0)
        pltpu.sync_copy(wb_v, w_hbm.at[pl.ds(base, CH)])
        return 0

    lax.fori_loop(0, NCH, _step, 0)


_norm_call = functools.partial(
    pl.kernel,
    out_type=jax.ShapeDtypeStruct((E,), jnp.float32),
    mesh=_MESH,
    scratch_types=[
        pltpu.VMEM((CH,), jnp.int32),
        pltpu.VMEM((CH,), jnp.float32),
        pltpu.VMEM((CH,), jnp.float32),
        pltpu.SemaphoreType.DMA,
    ],
)(_norm_body)


def _agg_body(allx_hbm, eidx_hbm, dst_hbm, w_hbm, out_hbm,
              ei0_v, ei1_v, d0_v, d1_v, w0_v, w1_v, r0_v, r1_v, agg_sh,
              se0, se1, sg0, sg1):
    c = lax.axis_index("c")
    s = lax.axis_index("s")
    w = _wid(c, s)

    def _zr(i, _):
        for j in range(D // 16):
            r0_v[i, pl.ds(j * 16, 16)] = _z16()
        return 0

    lax.fori_loop(0, ZR, _zr, 0)
    for k in range(NZB // NS + 1):
        b = s + NS * k

        @pl.when(b < NZB)
        def _():
            pltpu.sync_copy(r0_v, agg_sh.at[pl.ds(b * ZR, ZR)])

    plsc.subcore_barrier()

    def _fetch(i, ei_v, d_v, wc_v, sem):
        base = w * EPW + i * CH
        pltpu.async_copy(eidx_hbm.at[pl.ds(base, CH)], ei_v, sem)
        pltpu.async_copy(dst_hbm.at[pl.ds(base, CH)], d_v, sem)
        pltpu.async_copy(w_hbm.at[pl.ds(base, CH)], wc_v, sem)

    def _drain(i, ei_v, d_v, wc_v, sem):
        base = w * EPW + i * CH
        pltpu.make_async_copy(eidx_hbm.at[pl.ds(base, CH)], ei_v, sem).wait()
        pltpu.make_async_copy(dst_hbm.at[pl.ds(base, CH)], d_v, sem).wait()
        pltpu.make_async_copy(w_hbm.at[pl.ds(base, CH)], wc_v, sem).wait()

    def _proc(rows_v, wc_v, d_v):
        def _scale(g, _):
            wv = wc_v[pl.ds(g * 16, 16)]
            row0 = g * 16
            for b in range(16):
                wb = wv[b]
                for j in range(D // 16):
                    rows_v[row0 + b, pl.ds(j * 16, 16)] = (
                        rows_v[row0 + b, pl.ds(j * 16, 16)] * wb)
            return 0

        lax.fori_loop(0, CH // 16, _scale, 0)
        pltpu.sync_copy(rows_v, agg_sh.at[d_v], add=True)

    _fetch(0, ei0_v, d0_v, w0_v, se0)
    _fetch(1, ei1_v, d1_v, w1_v, se1)
    _drain(0, ei0_v, d0_v, w0_v, se0)
    pltpu.async_copy(allx_hbm.at[ei0_v], r0_v, sg0)

    def _pair(k, _):
        i0 = 2 * k
        i1 = i0 + 1
        _drain(i1, ei1_v, d1_v, w1_v, se1)
        pltpu.async_copy(allx_hbm.at[ei1_v], r1_v, sg1)
        pltpu.make_async_copy(allx_hbm.at[ei0_v], r0_v, sg0).wait()
        _proc(r0_v, w0_v, d0_v)
        _fetch(i0 + 2, ei0_v, d0_v, w0_v, se0)
        pltpu.make_async_copy(allx_hbm.at[ei1_v], r1_v, sg1).wait()
        _proc(r1_v, w1_v, d1_v)

        @pl.when(i1 + 2 < NCH)
        def _():
            _fetch(i1 + 2, ei1_v, d1_v, w1_v, se1)

        _drain(i0 + 2, ei0_v, d0_v, w0_v, se0)
        pltpu.async_copy(allx_hbm.at[ei0_v], r0_v, sg0)
        return 0

    lax.fori_loop(0, NPAIR, _pair, 0)
    pltpu.make_async_copy(allx_hbm.at[ei0_v], r0_v, sg0).wait()
    _proc(r0_v, w0_v, d0_v)

    plsc.subcore_barrier()

    for k in range(NZB // NS + 1):
        b = s + NS * k

        @pl.when(b < NZB)
        def _():
            pltpu.sync_copy(agg_sh.at[pl.ds(b * ZR, ZR)], r0_v)
            pltpu.sync_copy(r0_v, out_hbm.at[c, pl.ds(b * ZR, ZR)])


_agg_call = functools.partial(
    pl.kernel,
    out_type=jax.ShapeDtypeStruct((NC, N, D), jnp.float32),
    mesh=_MESH,
    scratch_types=[
        pltpu.VMEM((CH,), jnp.int32),
        pltpu.VMEM((CH,), jnp.int32),
        pltpu.VMEM((CH,), jnp.int32),
        pltpu.VMEM((CH,), jnp.int32),
        pltpu.VMEM((CH,), jnp.float32),
        pltpu.VMEM((CH,), jnp.float32),
        pltpu.VMEM((CH, D), jnp.float32),
        pltpu.VMEM((CH, D), jnp.float32),
        pltpu.VMEM_SHARED((N, D), jnp.float32),
        pltpu.SemaphoreType.DMA,
        pltpu.SemaphoreType.DMA,
        pltpu.SemaphoreType.DMA,
        pltpu.SemaphoreType.DMA,
    ],
)(_agg_body)


BLK = 1000
NBLK = N // BLK


def _first_body(x_ref, w_ref, ax_ref):
    xb = x_ref[...]
    for r in range(R):
        ax_ref[r] = jnp.dot(xb, w_ref[r], preferred_element_type=jnp.float32)


def _tc_first(x, W):
    out = pl.pallas_call(
        _first_body,
        grid=(NBLK,),
        in_specs=[
            pl.BlockSpec((BLK, D), lambda i: (i, 0)),
            pl.BlockSpec((R, D, D), lambda i: (0, 0, 0)),
        ],
        out_specs=pl.BlockSpec((R, BLK, D), lambda i: (0, i, 0)),
        out_shape=jax.ShapeDtypeStruct((R, N, D), jnp.float32),
    )(x, W)
    return out.reshape(SEG, D)


def _mid_body(p_ref, xp_ref, root_ref, b_ref, w_ref, h_ref, ax_ref):
    agg = p_ref[0] + p_ref[1]
    h = jnp.maximum(
        agg + jnp.dot(xp_ref[...], root_ref[...],
                      preferred_element_type=jnp.float32) + b_ref[...], 0.0)
    h_ref[...] = h
    for r in range(R):
        ax_ref[r] = jnp.dot(h, w_ref[r], preferred_element_type=jnp.float32)


def _tc_mid(parts, xp, root, b, Wn):
    h, ax = pl.pallas_call(
        _mid_body,
        grid=(NBLK,),
        in_specs=[
            pl.BlockSpec((NC, BLK, D), lambda i: (0, i, 0)),
            pl.BlockSpec((BLK, D), lambda i: (i, 0)),
            pl.BlockSpec((D, D), lambda i: (0, 0)),
            pl.BlockSpec((1, D), lambda i: (0, 0)),
            pl.BlockSpec((R, D, D), lambda i: (0, 0, 0)),
        ],
        out_specs=[
            pl.BlockSpec((BLK, D), lambda i: (i, 0)),
            pl.BlockSpec((R, BLK, D), lambda i: (0, i, 0)),
        ],
        out_shape=[
            jax.ShapeDtypeStruct((N, D), jnp.float32),
            jax.ShapeDtypeStruct((R, N, D), jnp.float32),
        ],
    )(parts, xp, root, b.reshape(1, D), Wn)
    return h, ax.reshape(SEG, D)


def _last_body(p_ref, xp_ref, root_ref, b_ref, fw_ref, o_ref):
    agg = p_ref[0] + p_ref[1]
    h = jnp.maximum(
        agg + jnp.dot(xp_ref[...], root_ref[...],
                      preferred_element_type=jnp.float32) + b_ref[...], 0.0)
    o_ref[...] = jnp.dot(h, fw_ref[...], preferred_element_type=jnp.float32)


def _tc_last(parts, xp, root, b, fw_pad):
    return pl.pallas_call(
        _last_body,
        grid=(NBLK,),
        in_specs=[
            pl.BlockSpec((NC, BLK, D), lambda i: (0, i, 0)),
            pl.BlockSpec((BLK, D), lambda i: (i, 0)),
            pl.BlockSpec((D, D), lambda i: (0, 0)),
            pl.BlockSpec((1, D), lambda i: (0, 0)),
            pl.BlockSpec((D, D), lambda i: (0, 0)),
        ],
        out_specs=pl.BlockSpec((BLK, D), lambda i: (i, 0)),
        out_shape=jax.ShapeDtypeStruct((N, D), jnp.float32),
    )(parts, xp, root, b.reshape(1, D), fw_pad)


def kernel(x, edge_index, edge_attr, W1, root1, b1, W2, root2, b2,
           W3, root3, b3, fc_w, fc_b):
    src = edge_index[0].astype(jnp.int32)
    dst = edge_index[1].astype(jnp.int32)
    et = edge_attr.astype(jnp.int32)
    comb = dst * R + et
    eidx = et * N + src

    cnt = _cnt_call(comb).reshape(NC, SEG).sum(axis=0)
    we = _norm_call(cnt, comb)

    ax = _tc_first(x, W1)
    parts = _agg_call(ax, eidx, dst, we)
    h1, ax = _tc_mid(parts, x, root1, b1, W2)
    parts = _agg_call(ax, eidx, dst, we)
    h2, ax = _tc_mid(parts, h1, root2, b2, W3)
    parts = _agg_call(ax, eidx, dst, we)
    fw_pad = jnp.pad(fc_w, ((0, 0), (0, D - fc_w.shape[1])))
    out = _tc_last(parts, h2, root3, b3, fw_pad)
    return out[:, :1] + fc_b


# speedup vs baseline: 10.8393x; 1.2326x over previous
import functools

import jax
import jax.numpy as jnp
from jax import lax
from jax.experimental import pallas as pl
from jax.experimental.pallas import tpu as pltpu
from jax.experimental.pallas import tpu_sc as plsc

N = 10000
E = 320000
D = 128
R = 16
SEG = N * R

NC = 2
NS = 16
NW = NC * NS
EPW = E // NW
CH = 80
NCH = EPW // CH
NPAIR = NCH // 2
SPW = SEG // NS
ZB = 2000
ZR = 80
NZB = N // ZR

_MESH = plsc.VectorSubcoreMesh(core_axis_name="c", subcore_axis_name="s")


def _z16():
    return jnp.zeros((16,), jnp.float32)


def _o16():
    return jnp.ones((16,), jnp.float32)


def _wid(c, s):
    return s * NC + c


def _cnt_body(comb_hbm, out_hbm, idx_v, ones_v, zb_v, cnt_sh):
    c = lax.axis_index("c")
    s = lax.axis_index("s")
    w = _wid(c, s)

    def _z(i, _):
        zb_v[pl.ds(i * 16, 16)] = _z16()
        return 0

    lax.fori_loop(0, ZB // 16, _z, 0)
    for k in range(CH // 16):
        ones_v[pl.ds(k * 16, 16)] = _o16()
    for k in range(SPW // ZB):
        pltpu.sync_copy(zb_v, cnt_sh.at[pl.ds(s * SPW + k * ZB, ZB)])
    plsc.subcore_barrier()

    def _step(i, _):
        pltpu.sync_copy(comb_hbm.at[pl.ds(w * EPW + i * CH, CH)], idx_v)
        pltpu.sync_copy(ones_v, cnt_sh.at[idx_v], add=True)
        return 0

    lax.fori_loop(0, NCH, _step, 0)
    plsc.subcore_barrier()
    for k in range(SPW // ZB):
        pltpu.sync_copy(cnt_sh.at[pl.ds(s * SPW + k * ZB, ZB)], zb_v)
        pltpu.sync_copy(zb_v, out_hbm.at[pl.ds(c * SEG + s * SPW + k * ZB,
                                               ZB)])


_cnt_call = functools.partial(
    pl.kernel,
    out_type=jax.ShapeDtypeStruct((NC * SEG,), jnp.float32),
    mesh=_MESH,
    scratch_types=[
        pltpu.VMEM((CH,), jnp.int32),
        pltpu.VMEM((CH,), jnp.float32),
        pltpu.VMEM((ZB,), jnp.float32),
        pltpu.VMEM_SHARED((SEG,), jnp.float32),
    ],
)(_cnt_body)


def _norm_body(cnt_hbm, comb_hbm, w_hbm, cb0_v, cb1_v, c0_v, c1_v,
               wb0_v, wb1_v, sf0, sf1, sn0, sn1, sw0, sw1):
    c = lax.axis_index("c")
    s = lax.axis_index("s")
    w = _wid(c, s)

    def _base(i):
        return w * EPW + i * CH

    def _compute(i, c_v, wb_v, swsem):
        for k in range(CH // 16):
            cv = c_v[pl.ds(k * 16, 16)]
            wb_v[pl.ds(k * 16, 16)] = 1.0 / jnp.maximum(cv, 1.0)
        pltpu.async_copy(wb_v, w_hbm.at[pl.ds(_base(i), CH)], swsem)

    pltpu.async_copy(comb_hbm.at[pl.ds(_base(0), CH)], cb0_v, sf0)
    pltpu.async_copy(comb_hbm.at[pl.ds(_base(1), CH)], cb1_v, sf1)
    pltpu.make_async_copy(comb_hbm.at[pl.ds(_base(0), CH)], cb0_v, sf0).wait()
    pltpu.async_copy(cnt_hbm.at[cb0_v], c0_v, sn0)

    def _pair(k, _):
        i0 = 2 * k
        i1 = i0 + 1

        @pl.when(k > 0)
        def _():
            pltpu.make_async_copy(wb1_v, w_hbm.at[pl.ds(_base(i1 - 2), CH)],
                                  sw1).wait()
            pltpu.make_async_copy(wb0_v, w_hbm.at[pl.ds(_base(i0 - 2), CH)],
                                  sw0).wait()

        pltpu.make_async_copy(comb_hbm.at[pl.ds(_base(i1), CH)], cb1_v,
                              sf1).wait()
        pltpu.async_copy(cnt_hbm.at[cb1_v], c1_v, sn1)
        pltpu.make_async_copy(cnt_hbm.at[cb0_v], c0_v, sn0).wait()
        _compute(i0, c0_v, wb0_v, sw0)
        pltpu.async_copy(comb_hbm.at[pl.ds(_base(i0 + 2), CH)], cb0_v, sf0)
        pltpu.make_async_copy(cnt_hbm.at[cb1_v], c1_v, sn1).wait()
        _compute(i1, c1_v, wb1_v, sw1)

        @pl.when(i1 + 2 < NCH)
        def _():
            pltpu.async_copy(comb_hbm.at[pl.ds(_base(i1 + 2), CH)], cb1_v,
                             sf1)

        pltpu.make_async_copy(comb_hbm.at[pl.ds(_base(i0 + 2), CH)], cb0_v,
                              sf0).wait()
        pltpu.async_copy(cnt_hbm.at[cb0_v], c0_v, sn0)
        return 0

    lax.fori_loop(0, NPAIR, _pair, 0)
    pltpu.make_async_copy(wb1_v, w_hbm.at[pl.ds(_base(NCH - 2), CH)],
                          sw1).wait()
    pltpu.make_async_copy(wb0_v, w_hbm.at[pl.ds(_base(NCH - 3), CH)],
                          sw0).wait()
    pltpu.make_async_copy(cnt_hbm.at[cb0_v], c0_v, sn0).wait()
    _compute(NCH - 1, c0_v, wb0_v, sw0)
    pltpu.make_async_copy(wb0_v, w_hbm.at[pl.ds(_base(NCH - 1), CH)],
                          sw0).wait()


_norm_call = functools.partial(
    pl.kernel,
    out_type=jax.ShapeDtypeStruct((E,), jnp.float32),
    mesh=_MESH,
    scratch_types=[
        pltpu.VMEM((CH,), jnp.int32),
        pltpu.VMEM((CH,), jnp.int32),
        pltpu.VMEM((CH,), jnp.float32),
        pltpu.VMEM((CH,), jnp.float32),
        pltpu.VMEM((CH,), jnp.float32),
        pltpu.VMEM((CH,), jnp.float32),
        pltpu.SemaphoreType.DMA,
        pltpu.SemaphoreType.DMA,
        pltpu.SemaphoreType.DMA,
        pltpu.SemaphoreType.DMA,
        pltpu.SemaphoreType.DMA,
        pltpu.SemaphoreType.DMA,
    ],
)(_norm_body)


def _agg_body(allx_hbm, eidx_hbm, dst_hbm, w_hbm, out_hbm,
              ei0_v, ei1_v, d0_v, d1_v, dsc0_v, dsc1_v, w0_v, w1_v,
              r0_v, r1_v, agg_sh, se0, se1, sg0, sg1, ss0, ss1):
    c = lax.axis_index("c")
    s = lax.axis_index("s")
    w = _wid(c, s)

    def _zr(i, _):
        for j in range(D // 16):
            r0_v[i, pl.ds(j * 16, 16)] = _z16()
        return 0

    lax.fori_loop(0, ZR, _zr, 0)
    for k in range(NZB // NS + 1):
        b = s + NS * k

        @pl.when(b < NZB)
        def _():
            pltpu.sync_copy(r0_v, agg_sh.at[pl.ds(b * ZR, ZR)])

    plsc.subcore_barrier()

    def _fetch(i, ei_v, d_v, wc_v, sem):
        base = w * EPW + i * CH
        pltpu.async_copy(eidx_hbm.at[pl.ds(base, CH)], ei_v, sem)
        pltpu.async_copy(dst_hbm.at[pl.ds(base, CH)], d_v, sem)
        pltpu.async_copy(w_hbm.at[pl.ds(base, CH)], wc_v, sem)

    def _drain(i, ei_v, d_v, wc_v, sem):
        base = w * EPW + i * CH
        pltpu.make_async_copy(eidx_hbm.at[pl.ds(base, CH)], ei_v, sem).wait()
        pltpu.make_async_copy(dst_hbm.at[pl.ds(base, CH)], d_v, sem).wait()
        pltpu.make_async_copy(w_hbm.at[pl.ds(base, CH)], wc_v, sem).wait()

    def _proc(rows_v, wc_v, d_v, dsc_v, ssem):
        def _scale(g, _):
            wv = wc_v[pl.ds(g * 16, 16)]
            row0 = g * 16
            for b in range(16):
                wb = wv[b]
                for j in range(D // 16):
                    rows_v[row0 + b, pl.ds(j * 16, 16)] = (
                        rows_v[row0 + b, pl.ds(j * 16, 16)] * wb)
            return 0

        lax.fori_loop(0, CH // 16, _scale, 0)
        for t in range(CH // 16):
            dsc_v[pl.ds(t * 16, 16)] = d_v[pl.ds(t * 16, 16)]
        pltpu.async_copy(rows_v, agg_sh.at[dsc_v], ssem, add=True)

    _fetch(0, ei0_v, d0_v, w0_v, se0)
    _fetch(1, ei1_v, d1_v, w1_v, se1)
    _drain(0, ei0_v, d0_v, w0_v, se0)
    pltpu.async_copy(allx_hbm.at[ei0_v], r0_v, sg0)

    def _pair(k, _):
        i0 = 2 * k
        i1 = i0 + 1

        @pl.when(k > 0)
        def _():
            pltpu.make_async_copy(r1_v, agg_sh.at[dsc1_v], ss1).wait()

        _drain(i1, ei1_v, d1_v, w1_v, se1)
        pltpu.async_copy(allx_hbm.at[ei1_v], r1_v, sg1)
        pltpu.make_async_copy(allx_hbm.at[ei0_v], r0_v, sg0).wait()
        _proc(r0_v, w0_v, d0_v, dsc0_v, ss0)
        _fetch(i0 + 2, ei0_v, d0_v, w0_v, se0)
        pltpu.make_async_copy(allx_hbm.at[ei1_v], r1_v, sg1).wait()
        _proc(r1_v, w1_v, d1_v, dsc1_v, ss1)

        @pl.when(i1 + 2 < NCH)
        def _():
            _fetch(i1 + 2, ei1_v, d1_v, w1_v, se1)

        pltpu.make_async_copy(r0_v, agg_sh.at[dsc0_v], ss0).wait()
        _drain(i0 + 2, ei0_v, d0_v, w0_v, se0)
        pltpu.async_copy(allx_hbm.at[ei0_v], r0_v, sg0)
        return 0

    lax.fori_loop(0, NPAIR, _pair, 0)
    pltpu.make_async_copy(r1_v, agg_sh.at[dsc1_v], ss1).wait()
    pltpu.make_async_copy(allx_hbm.at[ei0_v], r0_v, sg0).wait()
    _proc(r0_v, w0_v, d0_v, dsc0_v, ss0)
    pltpu.make_async_copy(r0_v, agg_sh.at[dsc0_v], ss0).wait()

    plsc.subcore_barrier()

    for k in range(NZB // NS + 1):
        b = s + NS * k

        @pl.when(b < NZB)
        def _():
            pltpu.sync_copy(agg_sh.at[pl.ds(b * ZR, ZR)], r0_v)
            pltpu.sync_copy(r0_v, out_hbm.at[c, pl.ds(b * ZR, ZR)])


_agg_call = functools.partial(
    pl.kernel,
    out_type=jax.ShapeDtypeStruct((NC, N, D), jnp.float32),
    mesh=_MESH,
    scratch_types=[
        pltpu.VMEM((CH,), jnp.int32),
        pltpu.VMEM((CH,), jnp.int32),
        pltpu.VMEM((CH,), jnp.int32),
        pltpu.VMEM((CH,), jnp.int32),
        pltpu.VMEM((CH,), jnp.int32),
        pltpu.VMEM((CH,), jnp.int32),
        pltpu.VMEM((CH,), jnp.float32),
        pltpu.VMEM((CH,), jnp.float32),
        pltpu.VMEM((CH, D), jnp.float32),
        pltpu.VMEM((CH, D), jnp.float32),
        pltpu.VMEM_SHARED((N, D), jnp.float32),
        pltpu.SemaphoreType.DMA,
        pltpu.SemaphoreType.DMA,
        pltpu.SemaphoreType.DMA,
        pltpu.SemaphoreType.DMA,
        pltpu.SemaphoreType.DMA,
        pltpu.SemaphoreType.DMA,
    ],
)(_agg_body)


BLK = 1000
NBLK = N // BLK


def _first_body(x_ref, w_ref, ax_ref):
    xb = x_ref[...]
    for r in range(R):
        ax_ref[r] = jnp.dot(xb, w_ref[r], preferred_element_type=jnp.float32)


def _tc_first(x, W):
    out = pl.pallas_call(
        _first_body,
        grid=(NBLK,),
        in_specs=[
            pl.BlockSpec((BLK, D), lambda i: (i, 0)),
            pl.BlockSpec((R, D, D), lambda i: (0, 0, 0)),
        ],
        out_specs=pl.BlockSpec((R, BLK, D), lambda i: (0, i, 0)),
        out_shape=jax.ShapeDtypeStruct((R, N, D), jnp.float32),
    )(x, W)
    return out.reshape(SEG, D)


def _mid_body(p_ref, xp_ref, root_ref, b_ref, w_ref, h_ref, ax_ref):
    agg = p_ref[0] + p_ref[1]
    h = jnp.maximum(
        agg + jnp.dot(xp_ref[...], root_ref[...],
                      preferred_element_type=jnp.float32) + b_ref[...], 0.0)
    h_ref[...] = h
    for r in range(R):
        ax_ref[r] = jnp.dot(h, w_ref[r], preferred_element_type=jnp.float32)


def _tc_mid(parts, xp, root, b, Wn):
    h, ax = pl.pallas_call(
        _mid_body,
        grid=(NBLK,),
        in_specs=[
            pl.BlockSpec((NC, BLK, D), lambda i: (0, i, 0)),
            pl.BlockSpec((BLK, D), lambda i: (i, 0)),
            pl.BlockSpec((D, D), lambda i: (0, 0)),
            pl.BlockSpec((1, D), lambda i: (0, 0)),
            pl.BlockSpec((R, D, D), lambda i: (0, 0, 0)),
        ],
        out_specs=[
            pl.BlockSpec((BLK, D), lambda i: (i, 0)),
            pl.BlockSpec((R, BLK, D), lambda i: (0, i, 0)),
        ],
        out_shape=[
            jax.ShapeDtypeStruct((N, D), jnp.float32),
            jax.ShapeDtypeStruct((R, N, D), jnp.float32),
        ],
    )(parts, xp, root, b.reshape(1, D), Wn)
    return h, ax.reshape(SEG, D)


def _last_body(p_ref, xp_ref, root_ref, b_ref, fw_ref, o_ref):
    agg = p_ref[0] + p_ref[1]
    h = jnp.maximum(
        agg + jnp.dot(xp_ref[...], root_ref[...],
                      preferred_element_type=jnp.float32) + b_ref[...], 0.0)
    o_ref[...] = jnp.dot(h, fw_ref[...], preferred_element_type=jnp.float32)


def _tc_last(parts, xp, root, b, fw_pad):
    return pl.pallas_call(
        _last_body,
        grid=(NBLK,),
        in_specs=[
            pl.BlockSpec((NC, BLK, D), lambda i: (0, i, 0)),
            pl.BlockSpec((BLK, D), lambda i: (i, 0)),
            pl.BlockSpec((D, D), lambda i: (0, 0)),
            pl.BlockSpec((1, D), lambda i: (0, 0)),
            pl.BlockSpec((D, D), lambda i: (0, 0)),
        ],
        out_specs=pl.BlockSpec((BLK, D), lambda i: (i, 0)),
        out_shape=jax.ShapeDtypeStruct((N, D), jnp.float32),
    )(parts, xp, root, b.reshape(1, D), fw_pad)


def kernel(x, edge_index, edge_attr, W1, root1, b1, W2, root2, b2,
           W3, root3, b3, fc_w, fc_b):
    src = edge_index[0].astype(jnp.int32)
    dst = edge_index[1].astype(jnp.int32)
    et = edge_attr.astype(jnp.int32)
    comb = dst * R + et
    eidx = et * N + src

    cnt = _cnt_call(comb).reshape(NC, SEG).sum(axis=0)
    we = _norm_call(cnt, comb)

    ax = _tc_first(x, W1)
    parts = _agg_call(ax, eidx, dst, we)
    h1, ax = _tc_mid(parts, x, root1, b1, W2)
    parts = _agg_call(ax, eidx, dst, we)
    h2, ax = _tc_mid(parts, h1, root2, b2, W3)
    parts = _agg_call(ax, eidx, dst, we)
    fw_pad = jnp.pad(fc_w, ((0, 0), (0, D - fc_w.shape[1])))
    out = _tc_last(parts, h2, root3, b3, fw_pad)
    return out[:, :1] + fc_b


# speedup vs baseline: 11.3940x; 1.0512x over previous
import functools

import jax
import jax.numpy as jnp
from jax import lax
from jax.experimental import pallas as pl
from jax.experimental.pallas import tpu as pltpu
from jax.experimental.pallas import tpu_sc as plsc

N = 10000
E = 320000
D = 128
R = 16
SEG = N * R

NC = 2
NS = 16
NW = NC * NS
EPW = E // NW
CH = 80
NCH = EPW // CH
NPAIR = NCH // 2
SPW = SEG // NS
ZB = 2000
ZR = 80
NZB = N // ZR

_MESH = plsc.VectorSubcoreMesh(core_axis_name="c", subcore_axis_name="s")


def _z16():
    return jnp.zeros((16,), jnp.float32)


def _o16():
    return jnp.ones((16,), jnp.float32)


def _wid(c, s):
    return s * NC + c


def _cnt_body(comb_hbm, out_hbm, cb0_v, cb1_v, csc0_v, csc1_v, ones_v,
              zb_v, cnt_sh, sf0, sf1, sc0, sc1):
    c = lax.axis_index("c")
    s = lax.axis_index("s")
    w = _wid(c, s)

    def _z(i, _):
        zb_v[pl.ds(i * 16, 16)] = _z16()
        return 0

    lax.fori_loop(0, ZB // 16, _z, 0)
    for k in range(CH // 16):
        ones_v[pl.ds(k * 16, 16)] = _o16()
    for k in range(SPW // ZB):
        pltpu.sync_copy(zb_v, cnt_sh.at[pl.ds(s * SPW + k * ZB, ZB)])
    plsc.subcore_barrier()

    def _base(i):
        return w * EPW + i * CH

    def _snap_scatter(cb_v, csc_v, ssem):
        for t in range(CH // 16):
            csc_v[pl.ds(t * 16, 16)] = cb_v[pl.ds(t * 16, 16)]
        pltpu.async_copy(ones_v, cnt_sh.at[csc_v], ssem, add=True)

    pltpu.async_copy(comb_hbm.at[pl.ds(_base(0), CH)], cb0_v, sf0)
    pltpu.async_copy(comb_hbm.at[pl.ds(_base(1), CH)], cb1_v, sf1)
    pltpu.make_async_copy(comb_hbm.at[pl.ds(_base(0), CH)], cb0_v, sf0).wait()

    def _pair(k, _):
        i0 = 2 * k
        i1 = i0 + 1

        @pl.when(k > 0)
        def _():
            pltpu.make_async_copy(ones_v, cnt_sh.at[csc0_v], sc0).wait()

        _snap_scatter(cb0_v, csc0_v, sc0)
        pltpu.async_copy(comb_hbm.at[pl.ds(_base(i0 + 2), CH)], cb0_v, sf0)
        pltpu.make_async_copy(comb_hbm.at[pl.ds(_base(i1), CH)], cb1_v,
                              sf1).wait()

        @pl.when(k > 0)
        def _():
            pltpu.make_async_copy(ones_v, cnt_sh.at[csc1_v], sc1).wait()

        _snap_scatter(cb1_v, csc1_v, sc1)

        @pl.when(i1 + 2 < NCH)
        def _():
            pltpu.async_copy(comb_hbm.at[pl.ds(_base(i1 + 2), CH)], cb1_v,
                             sf1)

        pltpu.make_async_copy(comb_hbm.at[pl.ds(_base(i0 + 2), CH)], cb0_v,
                              sf0).wait()
        return 0

    lax.fori_loop(0, NPAIR, _pair, 0)
    pltpu.make_async_copy(ones_v, cnt_sh.at[csc0_v], sc0).wait()
    _snap_scatter(cb0_v, csc0_v, sc0)
    pltpu.make_async_copy(ones_v, cnt_sh.at[csc0_v], sc0).wait()
    pltpu.make_async_copy(ones_v, cnt_sh.at[csc1_v], sc1).wait()
    plsc.subcore_barrier()
    for k in range(SPW // ZB):
        pltpu.sync_copy(cnt_sh.at[pl.ds(s * SPW + k * ZB, ZB)], zb_v)
        pltpu.sync_copy(zb_v, out_hbm.at[pl.ds(c * SEG + s * SPW + k * ZB,
                                               ZB)])


_cnt_call = functools.partial(
    pl.kernel,
    out_type=jax.ShapeDtypeStruct((NC * SEG,), jnp.float32),
    mesh=_MESH,
    scratch_types=[
        pltpu.VMEM((CH,), jnp.int32),
        pltpu.VMEM((CH,), jnp.int32),
        pltpu.VMEM((CH,), jnp.int32),
        pltpu.VMEM((CH,), jnp.int32),
        pltpu.VMEM((CH,), jnp.float32),
        pltpu.VMEM((ZB,), jnp.float32),
        pltpu.VMEM_SHARED((SEG,), jnp.float32),
        pltpu.SemaphoreType.DMA,
        pltpu.SemaphoreType.DMA,
        pltpu.SemaphoreType.DMA,
        pltpu.SemaphoreType.DMA,
    ],
)(_cnt_body)


def _norm_body(cnt_hbm, comb_hbm, w_hbm, cb0_v, cb1_v, c0_v, c1_v,
               wb0_v, wb1_v, sf0, sf1, sn0, sn1, sw0, sw1):
    c = lax.axis_index("c")
    s = lax.axis_index("s")
    w = _wid(c, s)

    def _base(i):
        return w * EPW + i * CH

    def _compute(i, c_v, wb_v, swsem):
        for k in range(CH // 16):
            cv = c_v[pl.ds(k * 16, 16)]
            wb_v[pl.ds(k * 16, 16)] = 1.0 / jnp.maximum(cv, 1.0)
        pltpu.async_copy(wb_v, w_hbm.at[pl.ds(_base(i), CH)], swsem)

    pltpu.async_copy(comb_hbm.at[pl.ds(_base(0), CH)], cb0_v, sf0)
    pltpu.async_copy(comb_hbm.at[pl.ds(_base(1), CH)], cb1_v, sf1)
    pltpu.make_async_copy(comb_hbm.at[pl.ds(_base(0), CH)], cb0_v, sf0).wait()
    pltpu.async_copy(cnt_hbm.at[cb0_v], c0_v, sn0)

    def _pair(k, _):
        i0 = 2 * k
        i1 = i0 + 1

        @pl.when(k > 0)
        def _():
            pltpu.make_async_copy(wb1_v, w_hbm.at[pl.ds(_base(i1 - 2), CH)],
                                  sw1).wait()
            pltpu.make_async_copy(wb0_v, w_hbm.at[pl.ds(_base(i0 - 2), CH)],
                                  sw0).wait()

        pltpu.make_async_copy(comb_hbm.at[pl.ds(_base(i1), CH)], cb1_v,
                              sf1).wait()
        pltpu.async_copy(cnt_hbm.at[cb1_v], c1_v, sn1)
        pltpu.make_async_copy(cnt_hbm.at[cb0_v], c0_v, sn0).wait()
        _compute(i0, c0_v, wb0_v, sw0)
        pltpu.async_copy(comb_hbm.at[pl.ds(_base(i0 + 2), CH)], cb0_v, sf0)
        pltpu.make_async_copy(cnt_hbm.at[cb1_v], c1_v, sn1).wait()
        _compute(i1, c1_v, wb1_v, sw1)

        @pl.when(i1 + 2 < NCH)
        def _():
            pltpu.async_copy(comb_hbm.at[pl.ds(_base(i1 + 2), CH)], cb1_v,
                             sf1)

        pltpu.make_async_copy(comb_hbm.at[pl.ds(_base(i0 + 2), CH)], cb0_v,
                              sf0).wait()
        pltpu.async_copy(cnt_hbm.at[cb0_v], c0_v, sn0)
        return 0

    lax.fori_loop(0, NPAIR, _pair, 0)
    pltpu.make_async_copy(wb1_v, w_hbm.at[pl.ds(_base(NCH - 2), CH)],
                          sw1).wait()
    pltpu.make_async_copy(wb0_v, w_hbm.at[pl.ds(_base(NCH - 3), CH)],
                          sw0).wait()
    pltpu.make_async_copy(cnt_hbm.at[cb0_v], c0_v, sn0).wait()
    _compute(NCH - 1, c0_v, wb0_v, sw0)
    pltpu.make_async_copy(wb0_v, w_hbm.at[pl.ds(_base(NCH - 1), CH)],
                          sw0).wait()


_norm_call = functools.partial(
    pl.kernel,
    out_type=jax.ShapeDtypeStruct((E,), jnp.float32),
    mesh=_MESH,
    scratch_types=[
        pltpu.VMEM((CH,), jnp.int32),
        pltpu.VMEM((CH,), jnp.int32),
        pltpu.VMEM((CH,), jnp.float32),
        pltpu.VMEM((CH,), jnp.float32),
        pltpu.VMEM((CH,), jnp.float32),
        pltpu.VMEM((CH,), jnp.float32),
        pltpu.SemaphoreType.DMA,
        pltpu.SemaphoreType.DMA,
        pltpu.SemaphoreType.DMA,
        pltpu.SemaphoreType.DMA,
        pltpu.SemaphoreType.DMA,
        pltpu.SemaphoreType.DMA,
    ],
)(_norm_body)


def _agg_body(allx_hbm, eidx_hbm, dst_hbm, w_hbm, out_hbm,
              ei0_v, ei1_v, d0_v, d1_v, dsc0_v, dsc1_v, w0_v, w1_v,
              r0_v, r1_v, agg_sh, se0, se1, sg0, sg1, ss0, ss1):
    c = lax.axis_index("c")
    s = lax.axis_index("s")
    w = _wid(c, s)

    def _zr(i, _):
        for j in range(D // 16):
            r0_v[i, pl.ds(j * 16, 16)] = _z16()
        return 0

    lax.fori_loop(0, ZR, _zr, 0)
    for k in range(NZB // NS + 1):
        b = s + NS * k

        @pl.when(b < NZB)
        def _():
            pltpu.sync_copy(r0_v, agg_sh.at[pl.ds(b * ZR, ZR)])

    plsc.subcore_barrier()

    def _fetch(i, ei_v, d_v, wc_v, sem):
        base = w * EPW + i * CH
        pltpu.async_copy(eidx_hbm.at[pl.ds(base, CH)], ei_v, sem)
        pltpu.async_copy(dst_hbm.at[pl.ds(base, CH)], d_v, sem)
        pltpu.async_copy(w_hbm.at[pl.ds(base, CH)], wc_v, sem)

    def _drain(i, ei_v, d_v, wc_v, sem):
        base = w * EPW + i * CH
        pltpu.make_async_copy(eidx_hbm.at[pl.ds(base, CH)], ei_v, sem).wait()
        pltpu.make_async_copy(dst_hbm.at[pl.ds(base, CH)], d_v, sem).wait()
        pltpu.make_async_copy(w_hbm.at[pl.ds(base, CH)], wc_v, sem).wait()

    def _proc(rows_v, wc_v, d_v, dsc_v, ssem):
        def _scale(g, _):
            wv = wc_v[pl.ds(g * 16, 16)]
            row0 = g * 16
            for b in range(16):
                wb = wv[b]
                for j in range(D // 16):
                    rows_v[row0 + b, pl.ds(j * 16, 16)] = (
                        rows_v[row0 + b, pl.ds(j * 16, 16)] * wb)
            return 0

        lax.fori_loop(0, CH // 16, _scale, 0)
        for t in range(CH // 16):
            dsc_v[pl.ds(t * 16, 16)] = d_v[pl.ds(t * 16, 16)]
        pltpu.async_copy(rows_v, agg_sh.at[dsc_v], ssem, add=True)

    _fetch(0, ei0_v, d0_v, w0_v, se0)
    _fetch(1, ei1_v, d1_v, w1_v, se1)
    _drain(0, ei0_v, d0_v, w0_v, se0)
    pltpu.async_copy(allx_hbm.at[ei0_v], r0_v, sg0)

    def _pair(k, _):
        i0 = 2 * k
        i1 = i0 + 1

        @pl.when(k > 0)
        def _():
            pltpu.make_async_copy(r1_v, agg_sh.at[dsc1_v], ss1).wait()

        _drain(i1, ei1_v, d1_v, w1_v, se1)
        pltpu.async_copy(allx_hbm.at[ei1_v], r1_v, sg1)
        pltpu.make_async_copy(allx_hbm.at[ei0_v], r0_v, sg0).wait()
        _proc(r0_v, w0_v, d0_v, dsc0_v, ss0)
        _fetch(i0 + 2, ei0_v, d0_v, w0_v, se0)
        pltpu.make_async_copy(allx_hbm.at[ei1_v], r1_v, sg1).wait()
        _proc(r1_v, w1_v, d1_v, dsc1_v, ss1)

        @pl.when(i1 + 2 < NCH)
        def _():
            _fetch(i1 + 2, ei1_v, d1_v, w1_v, se1)

        pltpu.make_async_copy(r0_v, agg_sh.at[dsc0_v], ss0).wait()
        _drain(i0 + 2, ei0_v, d0_v, w0_v, se0)
        pltpu.async_copy(allx_hbm.at[ei0_v], r0_v, sg0)
        return 0

    lax.fori_loop(0, NPAIR, _pair, 0)
    pltpu.make_async_copy(r1_v, agg_sh.at[dsc1_v], ss1).wait()
    pltpu.make_async_copy(allx_hbm.at[ei0_v], r0_v, sg0).wait()
    _proc(r0_v, w0_v, d0_v, dsc0_v, ss0)
    pltpu.make_async_copy(r0_v, agg_sh.at[dsc0_v], ss0).wait()

    plsc.subcore_barrier()

    for k in range(NZB // NS + 1):
        b = s + NS * k

        @pl.when(b < NZB)
        def _():
            pltpu.sync_copy(agg_sh.at[pl.ds(b * ZR, ZR)], r0_v)
            pltpu.sync_copy(r0_v, out_hbm.at[c, pl.ds(b * ZR, ZR)])


_agg_call = functools.partial(
    pl.kernel,
    out_type=jax.ShapeDtypeStruct((NC, N, D), jnp.float32),
    mesh=_MESH,
    scratch_types=[
        pltpu.VMEM((CH,), jnp.int32),
        pltpu.VMEM((CH,), jnp.int32),
        pltpu.VMEM((CH,), jnp.int32),
        pltpu.VMEM((CH,), jnp.int32),
        pltpu.VMEM((CH,), jnp.int32),
        pltpu.VMEM((CH,), jnp.int32),
        pltpu.VMEM((CH,), jnp.float32),
        pltpu.VMEM((CH,), jnp.float32),
        pltpu.VMEM((CH, D), jnp.float32),
        pltpu.VMEM((CH, D), jnp.float32),
        pltpu.VMEM_SHARED((N, D), jnp.float32),
        pltpu.SemaphoreType.DMA,
        pltpu.SemaphoreType.DMA,
        pltpu.SemaphoreType.DMA,
        pltpu.SemaphoreType.DMA,
        pltpu.SemaphoreType.DMA,
        pltpu.SemaphoreType.DMA,
    ],
)(_agg_body)


BLK = 1000
NBLK = N // BLK


def _first_body(x_ref, w_ref, ax_ref):
    xb = x_ref[...]
    for r in range(R):
        ax_ref[r] = jnp.dot(xb, w_ref[r], preferred_element_type=jnp.float32)


def _tc_first(x, W):
    out = pl.pallas_call(
        _first_body,
        grid=(NBLK,),
        in_specs=[
            pl.BlockSpec((BLK, D), lambda i: (i, 0)),
            pl.BlockSpec((R, D, D), lambda i: (0, 0, 0)),
        ],
        out_specs=pl.BlockSpec((R, BLK, D), lambda i: (0, i, 0)),
        out_shape=jax.ShapeDtypeStruct((R, N, D), jnp.float32),
    )(x, W)
    return out.reshape(SEG, D)


def _mid_body(p_ref, xp_ref, root_ref, b_ref, w_ref, h_ref, ax_ref):
    agg = p_ref[0] + p_ref[1]
    h = jnp.maximum(
        agg + jnp.dot(xp_ref[...], root_ref[...],
                      preferred_element_type=jnp.float32) + b_ref[...], 0.0)
    h_ref[...] = h
    for r in range(R):
        ax_ref[r] = jnp.dot(h, w_ref[r], preferred_element_type=jnp.float32)


def _tc_mid(parts, xp, root, b, Wn):
    h, ax = pl.pallas_call(
        _mid_body,
        grid=(NBLK,),
        in_specs=[
            pl.BlockSpec((NC, BLK, D), lambda i: (0, i, 0)),
            pl.BlockSpec((BLK, D), lambda i: (i, 0)),
            pl.BlockSpec((D, D), lambda i: (0, 0)),
            pl.BlockSpec((1, D), lambda i: (0, 0)),
            pl.BlockSpec((R, D, D), lambda i: (0, 0, 0)),
        ],
        out_specs=[
            pl.BlockSpec((BLK, D), lambda i: (i, 0)),
            pl.BlockSpec((R, BLK, D), lambda i: (0, i, 0)),
        ],
        out_shape=[
            jax.ShapeDtypeStruct((N, D), jnp.float32),
            jax.ShapeDtypeStruct((R, N, D), jnp.float32),
        ],
    )(parts, xp, root, b.reshape(1, D), Wn)
    return h, ax.reshape(SEG, D)


def _last_body(p_ref, xp_ref, root_ref, b_ref, fw_ref, o_ref):
    agg = p_ref[0] + p_ref[1]
    h = jnp.maximum(
        agg + jnp.dot(xp_ref[...], root_ref[...],
                      preferred_element_type=jnp.float32) + b_ref[...], 0.0)
    o_ref[...] = jnp.dot(h, fw_ref[...], preferred_element_type=jnp.float32)


def _tc_last(parts, xp, root, b, fw_pad):
    return pl.pallas_call(
        _last_body,
        grid=(NBLK,),
        in_specs=[
            pl.BlockSpec((NC, BLK, D), lambda i: (0, i, 0)),
            pl.BlockSpec((BLK, D), lambda i: (i, 0)),
            pl.BlockSpec((D, D), lambda i: (0, 0)),
            pl.BlockSpec((1, D), lambda i: (0, 0)),
            pl.BlockSpec((D, D), lambda i: (0, 0)),
        ],
        out_specs=pl.BlockSpec((BLK, D), lambda i: (i, 0)),
        out_shape=jax.ShapeDtypeStruct((N, D), jnp.float32),
    )(parts, xp, root, b.reshape(1, D), fw_pad)


def kernel(x, edge_index, edge_attr, W1, root1, b1, W2, root2, b2,
           W3, root3, b3, fc_w, fc_b):
    src = edge_index[0].astype(jnp.int32)
    dst = edge_index[1].astype(jnp.int32)
    et = edge_attr.astype(jnp.int32)
    comb = dst * R + et
    eidx = et * N + src

    cnt = _cnt_call(comb).reshape(NC, SEG).sum(axis=0)
    we = _norm_call(cnt, comb)

    ax = _tc_first(x, W1)
    parts = _agg_call(ax, eidx, dst, we)
    h1, ax = _tc_mid(parts, x, root1, b1, W2)
    parts = _agg_call(ax, eidx, dst, we)
    h2, ax = _tc_mid(parts, h1, root2, b2, W3)
    parts = _agg_call(ax, eidx, dst, we)
    fw_pad = jnp.pad(fc_w, ((0, 0), (0, D - fc_w.shape[1])))
    out = _tc_last(parts, h2, root3, b3, fw_pad)
    return out[:, :1] + fc_b
